# baseline (speedup 1.0000x reference)
import jax
import jax.numpy as jnp
from jax.experimental import pallas as pl
from jax.experimental.pallas import tpu as pltpu

_N_FIELDS, _N_NODES, _D_FEAT = 2, 10000, 512


def _copy_body(src_ref, out_ref, sem):
    copy = pltpu.make_async_copy(src_ref, out_ref, sem)
    copy.start()
    copy.wait()


def kernel(old_g_nodes, new_g_nodes, time_map_nodes, weight, bias):
    out = pl.pallas_call(
        _copy_body,
        in_specs=[pl.BlockSpec(memory_space=pl.ANY)],
        out_specs=pl.BlockSpec(memory_space=pl.ANY),
        out_shape=jax.ShapeDtypeStruct(
            (_N_FIELDS, _N_NODES, _D_FEAT), jnp.float32
        ),
        scratch_shapes=[pltpu.SemaphoreType.DMA],
    )(new_g_nodes)
    return out


# speedup vs baseline: 42.1307x; 42.1307x over previous
import jax
import jax.numpy as jnp
from jax.experimental import pallas as pl

_N_FIELDS, _N_NODES, _D_FEAT = 2, 10000, 512
_BLOCK_ROWS = 1000


def _copy_body(src_ref, out_ref):
    out_ref[...] = src_ref[...]


def kernel(old_g_nodes, new_g_nodes, time_map_nodes, weight, bias):
    rows = _N_FIELDS * _N_NODES
    x = new_g_nodes.reshape(rows, _D_FEAT)
    out = pl.pallas_call(
        _copy_body,
        grid=(rows // _BLOCK_ROWS,),
        in_specs=[pl.BlockSpec((_BLOCK_ROWS, _D_FEAT), lambda i: (i, 0))],
        out_specs=pl.BlockSpec((_BLOCK_ROWS, _D_FEAT), lambda i: (i, 0)),
        out_shape=jax.ShapeDtypeStruct((rows, _D_FEAT), jnp.float32),
    )(x)
    return out.reshape(_N_FIELDS, _N_NODES, _D_FEAT)


# speedup vs baseline: 48.5793x; 1.1531x over previous
import jax
import jax.numpy as jnp
from jax.experimental import pallas as pl

_N_FIELDS, _N_NODES, _D_FEAT = 2, 10000, 512
_BLOCK_ROWS = 4000


def _copy_body(src_ref, out_ref):
    out_ref[...] = src_ref[...]


def kernel(old_g_nodes, new_g_nodes, time_map_nodes, weight, bias):
    rows = _N_FIELDS * _N_NODES
    x = new_g_nodes.reshape(rows, _D_FEAT)
    out = pl.pallas_call(
        _copy_body,
        grid=(rows // _BLOCK_ROWS,),
        in_specs=[pl.BlockSpec((_BLOCK_ROWS, _D_FEAT), lambda i: (i, 0))],
        out_specs=pl.BlockSpec((_BLOCK_ROWS, _D_FEAT), lambda i: (i, 0)),
        out_shape=jax.ShapeDtypeStruct((rows, _D_FEAT), jnp.float32),
    )(x)
    return out.reshape(_N_FIELDS, _N_NODES, _D_FEAT)


# speedup vs baseline: 48.8655x; 1.0059x over previous
import jax
import jax.numpy as jnp
from jax.experimental import pallas as pl

_N_FIELDS, _N_NODES, _D_FEAT = 2, 10000, 512
_BLOCK_ROWS = 5000


def _copy_body(src_ref, out_ref):
    out_ref[...] = src_ref[...]


def kernel(old_g_nodes, new_g_nodes, time_map_nodes, weight, bias):
    rows = _N_FIELDS * _N_NODES
    x = new_g_nodes.reshape(rows, _D_FEAT)
    out = pl.pallas_call(
        _copy_body,
        grid=(rows // _BLOCK_ROWS,),
        in_specs=[pl.BlockSpec((_BLOCK_ROWS, _D_FEAT), lambda i: (i, 0))],
        out_specs=pl.BlockSpec((_BLOCK_ROWS, _D_FEAT), lambda i: (i, 0)),
        out_shape=jax.ShapeDtypeStruct((rows, _D_FEAT), jnp.float32),
    )(x)
    return out.reshape(_N_FIELDS, _N_NODES, _D_FEAT)
